# baseline (speedup 1.0000x reference)
import functools

import jax
import jax.numpy as jnp
from jax.experimental import pallas as pl
from jax.experimental.pallas import tpu as pltpu


def _key_of(s):
    u = jax.lax.bitcast_convert_type(s, jnp.uint32)
    return jnp.where((u >> 31) != 0, ~u, u | jnp.uint32(0x80000000))


def _fused_body(gate_ref, ctx_ref, cond_ref, mtok_ref, out_ref, *, k):
    x = ctx_ref[0]
    c = cond_ref[0]
    n = x.shape[0]

    d = x.shape[1]
    g = n // 512
    x8 = x.reshape(g, 512, d)
    cn = c / jnp.maximum(jnp.sqrt(jnp.sum(c * c)), 1e-6)
    cnb = jnp.broadcast_to(cn, (g, d))
    dot = jax.lax.dot_general(
        x8, cnb, (((2,), (1,)), ((0,), (0,))),
        preferred_element_type=jnp.float32,
        precision=jax.lax.Precision.HIGHEST)
    ones = jnp.ones((g, d), jnp.float32)
    n2 = jax.lax.dot_general(
        x8 * x8, ones, (((2,), (1,)), ((0,), (0,))),
        preferred_element_type=jnp.float32,
        precision=jax.lax.Precision.HIGHEST)
    s = dot / jnp.maximum(jnp.sqrt(n2), 1e-6)

    key = _key_of(s)

    def bit_step(j, t):
        cand = t | (jnp.uint32(1) << (jnp.uint32(31) - j.astype(jnp.uint32)))
        cnt = jnp.sum((key >= cand).astype(jnp.int32))
        return jnp.where(cnt >= k, cand, t)

    t_kth = jax.lax.fori_loop(0, 32, bit_step, jnp.uint32(0))

    eq = key == t_kth
    r = k - jnp.sum((key > t_kth).astype(jnp.int32))
    idx = (jax.lax.broadcasted_iota(jnp.int32, (g, 512), 0) * 512
           + jax.lax.broadcasted_iota(jnp.int32, (g, 512), 1))

    def cut_search(_):
        def cut_step(j, lohi):
            lo, hi = lohi
            mid = (lo + hi) // 2
            cnt = jnp.sum((eq & (idx < mid)).astype(jnp.int32))
            return (jnp.where(cnt >= r, lo, mid + 1),
                    jnp.where(cnt >= r, mid, hi))
        return jax.lax.fori_loop(
            0, 13, cut_step, (jnp.int32(0), jnp.int32(n)))[1]

    eq_cnt = jnp.sum(eq.astype(jnp.int32))
    cut = jax.lax.cond(eq_cnt == r, lambda _: jnp.int32(n), cut_search,
                       jnp.int32(0))

    sel = (key > t_kth) | (eq & (idx < cut))
    sel = jnp.logical_and(sel, gate_ref[0, 0] != 0)
    sel8 = sel.astype(jnp.float32)[:, :, None]
    mtok = mtok_ref[...].reshape(1, 1, d)
    out_ref[0] = jnp.where(sel8 > 0.0, mtok, x8).reshape(n, d)


def kernel(ctx_tokens, cond_feat, mask_token, mask_ratio):
    B, N, D = ctx_tokens.shape
    k = max(1, int(0.25 * N))
    x = ctx_tokens.astype(jnp.float32)
    cond = cond_feat.astype(jnp.float32).reshape(B, 1, D)
    mtok = mask_token.astype(ctx_tokens.dtype).reshape(1, D)
    gate = (jnp.asarray(mask_ratio, jnp.float32) > 0).astype(
        jnp.int32).reshape(1, 1)

    body = functools.partial(_fused_body, k=k)
    out = pl.pallas_call(
        body,
        grid=(B,),
        in_specs=[
            pl.BlockSpec((1, 1), lambda b: (0, 0), memory_space=pltpu.SMEM),
            pl.BlockSpec((1, N, D), lambda b: (b, 0, 0)),
            pl.BlockSpec((1, 1, D), lambda b: (b, 0, 0)),
            pl.BlockSpec((1, D), lambda b: (0, 0)),
        ],
        out_specs=pl.BlockSpec((1, N, D), lambda b: (b, 0, 0)),
        out_shape=jax.ShapeDtypeStruct((B, N, D), ctx_tokens.dtype),
    )(gate, x, cond, mtok)
    return out


# speedup vs baseline: 3.3052x; 3.3052x over previous
import functools

import jax
import jax.numpy as jnp
from jax.experimental import pallas as pl
from jax.experimental.pallas import tpu as pltpu


def _key_of(s):
    u = jax.lax.bitcast_convert_type(s, jnp.uint32)
    return jnp.where((u >> 31) != 0, ~u, u | jnp.uint32(0x80000000))


def _fused_body(gate_ref, ctx_ref, cond_ref, mtok_ref, out_ref, *, k):
    x = ctx_ref[0]
    c = cond_ref[0]
    n = x.shape[0]

    cn = c / jnp.maximum(jnp.sqrt(jnp.sum(c * c)), 1e-6)
    dot = jax.lax.dot_general(
        x, cn, (((1,), (1,)), ((), ())),
        preferred_element_type=jnp.float32,
        precision=jax.lax.Precision.HIGHEST)
    ones = jnp.ones((1, x.shape[1]), jnp.float32)
    n2 = jax.lax.dot_general(
        x * x, ones, (((1,), (1,)), ((), ())),
        preferred_element_type=jnp.float32,
        precision=jax.lax.Precision.HIGHEST)
    s = dot.T / jnp.maximum(jnp.sqrt(n2.T), 1e-6)

    key = _key_of(s)

    def bit_step(j, t):
        sh = jnp.uint32(30) - 2 * j.astype(jnp.uint32)
        bh = jnp.uint32(2) << sh
        bl = jnp.uint32(1) << sh
        c3 = t | bh | bl
        c2 = t | bh
        c1 = t | bl
        n3 = jnp.sum((key >= c3).astype(jnp.int32))
        n2_ = jnp.sum((key >= c2).astype(jnp.int32))
        n1 = jnp.sum((key >= c1).astype(jnp.int32))
        return jnp.where(n3 >= k, c3,
                         jnp.where(n2_ >= k, c2,
                                   jnp.where(n1 >= k, c1, t)))

    t_kth = jax.lax.fori_loop(0, 16, bit_step, jnp.uint32(0))

    eq = key == t_kth
    r = k - jnp.sum((key > t_kth).astype(jnp.int32))
    idx = jax.lax.broadcasted_iota(jnp.int32, (1, n), 1)

    def cut_search(_):
        def cut_step(j, lohi):
            lo, hi = lohi
            mid = (lo + hi) // 2
            cnt = jnp.sum((eq & (idx < mid)).astype(jnp.int32))
            return (jnp.where(cnt >= r, lo, mid + 1),
                    jnp.where(cnt >= r, mid, hi))
        return jax.lax.fori_loop(
            0, 13, cut_step, (jnp.int32(0), jnp.int32(n)))[1]

    eq_cnt = jnp.sum(eq.astype(jnp.int32))
    cut = jax.lax.cond(eq_cnt == r, lambda _: jnp.int32(n), cut_search,
                       jnp.int32(0))

    sel = (key > t_kth) | (eq & (idx < cut))
    sel = jnp.logical_and(sel, gate_ref[0, 0] != 0)
    sel_col = sel.astype(jnp.float32).T
    out_ref[0] = jnp.where(sel_col > 0.0, mtok_ref[...], x)


def kernel(ctx_tokens, cond_feat, mask_token, mask_ratio):
    B, N, D = ctx_tokens.shape
    k = max(1, int(0.25 * N))
    x = ctx_tokens.astype(jnp.float32)
    cond = cond_feat.astype(jnp.float32).reshape(B, 1, D)
    mtok = mask_token.astype(ctx_tokens.dtype).reshape(1, D)
    gate = (jnp.asarray(mask_ratio, jnp.float32) > 0).astype(
        jnp.int32).reshape(1, 1)

    body = functools.partial(_fused_body, k=k)
    out = pl.pallas_call(
        body,
        grid=(B,),
        in_specs=[
            pl.BlockSpec((1, 1), lambda b: (0, 0), memory_space=pltpu.SMEM),
            pl.BlockSpec((1, N, D), lambda b: (b, 0, 0)),
            pl.BlockSpec((1, 1, D), lambda b: (b, 0, 0)),
            pl.BlockSpec((1, D), lambda b: (0, 0)),
        ],
        out_specs=pl.BlockSpec((1, N, D), lambda b: (b, 0, 0)),
        out_shape=jax.ShapeDtypeStruct((B, N, D), ctx_tokens.dtype),
    )(gate, x, cond, mtok)
    return out
